# baseline (speedup 1.0000x reference)
import functools

import jax
import jax.numpy as jnp
from jax import lax
from jax.experimental import pallas as pl
from jax.experimental.pallas import tpu as pltpu
from jax.experimental.pallas import tpu_sc as plsc

SCALE = 8.0

Q = 4


@functools.lru_cache(maxsize=None)
def _build(S, P, V, D):
    info = plsc.get_sparse_core_info()
    nw = info.num_cores * info.num_subcores
    n_chunks = (S // 128) * P
    cw = n_chunks // nw
    sh_n = S // 128
    n_iters = cw // Q

    @functools.partial(
        pl.kernel,
        out_type=jax.ShapeDtypeStruct((n_chunks * (D // 8), 8 * 128), jnp.float32),
        mesh=plsc.VectorSubcoreMesh(core_axis_name="c", subcore_axis_name="s"),
        compiler_params=pltpu.CompilerParams(
            use_tc_tiling_on_sc=False, needs_layout_passes=False
        ),
        scratch_types=[
            pltpu.VMEM((cw, 128), jnp.int32),
            pltpu.VMEM((Q * 128, D), jnp.float32),
            pltpu.VMEM((Q * D * 128,), jnp.float32),
            pltpu.SemaphoreType.DMA,
            pltpu.SemaphoreType.DMA,
        ],
    )
    def emb(tok_hbm, table_hbm, out_hbm, idx_v, buf, tbuf, gsem, ssem):
        wid = lax.axis_index("s") * info.num_cores + lax.axis_index("c")
        gid0 = wid * cw
        pltpu.sync_copy(tok_hbm.at[pl.ds(gid0, cw)], idx_v)
        iota = lax.iota(jnp.int32, 16)

        def it_body(t, carry):
            q0 = t * Q
            gathers = [
                pltpu.async_copy(
                    table_hbm.at[idx_v.at[q0 + q]],
                    buf.at[pl.ds(q * 128, 128)],
                    gsem,
                )
                for q in range(Q)
            ]
            for g in gathers:
                g.wait()

            def gr_body(gr, c2):
                colv = jnp.full((16,), gr, jnp.int32)
                for q in range(Q):
                    for l in range(8):
                        rowv = iota + (q * 128 + 16 * l)
                        vec = plsc.load_gather(buf, [rowv, colv])
                        tbuf[pl.ds(q * (D * 128) + gr * 128 + 16 * l, 16)] = (
                            vec * SCALE
                        )
                return c2

            lax.fori_loop(0, D, gr_body, 0)

            stores = []
            for q in range(Q):
                gid = gid0 + q0 + q
                p_lo = gid % 8
                ph_sh = gid // 8
                sh = ph_sh % sh_n
                p = 8 * (ph_sh // sh_n) + p_lo
                cb = p * (D // 8) * sh_n + sh
                for g in range(D // 8):
                    stores.append(
                        pltpu.async_copy(
                            tbuf.at[pl.ds(q * (D * 128) + g * 1024, 1024)],
                            out_hbm.at[cb + g * sh_n],
                            ssem,
                        )
                    )
            for st in stores:
                st.wait()
            return carry

        lax.fori_loop(0, n_iters, it_body, 0)

    return emb


def kernel(tokens, table):
    S, P = tokens.shape
    V, D = table.shape
    tok2 = (
        tokens.T.reshape(P // 8, 8, S // 128, 128)
        .transpose(0, 2, 1, 3)
        .reshape((S // 128) * P, 128)
        .astype(jnp.int32)
    )
    a2 = _build(S, P, V, D)(tok2, table)
    out = (
        a2.reshape(P, D // 8, S // 128, 8, 128)
        .transpose(2, 4, 0, 1, 3)
        .reshape(S, P, D)
    )
    return out


# speedup vs baseline: 1.6929x; 1.6929x over previous
import functools

import jax
import jax.numpy as jnp
from jax import lax
from jax.experimental import pallas as pl
from jax.experimental.pallas import tpu as pltpu
from jax.experimental.pallas import tpu_sc as plsc

SCALE = 8.0

Q = 4


@functools.lru_cache(maxsize=None)
def _build(S, P, V, D):
    info = plsc.get_sparse_core_info()
    nw = info.num_cores * info.num_subcores
    n_chunks = (S // 128) * P
    cw = n_chunks // nw
    sh_n = S // 128
    n_iters = cw // Q

    @functools.partial(
        pl.kernel,
        out_type=jax.ShapeDtypeStruct((n_chunks * (D // 8), 8, 128), jnp.float32),
        mesh=plsc.VectorSubcoreMesh(core_axis_name="c", subcore_axis_name="s"),
        compiler_params=pltpu.CompilerParams(
            use_tc_tiling_on_sc=False, needs_layout_passes=False
        ),
        scratch_types=[
            pltpu.VMEM((cw, 128), jnp.int32),
            pltpu.VMEM((Q * 128, D), jnp.float32),
            pltpu.VMEM((Q * D, 129), jnp.float32),
            pltpu.SemaphoreType.DMA,
            pltpu.SemaphoreType.DMA,
        ],
    )
    def emb(tok_hbm, table_hbm, out_hbm, idx_v, buf, tbuf, gsem, ssem):
        wid = lax.axis_index("s") * info.num_cores + lax.axis_index("c")
        gid0 = wid * cw
        pltpu.sync_copy(tok_hbm.at[pl.ds(gid0, cw)], idx_v)
        iota = lax.iota(jnp.int32, 16)
        rowvecs = [
            [iota + (q * D + 16 * m) for m in range(D // 16)] for q in range(Q)
        ]

        def it_body(t, carry):
            q0 = t * Q
            gathers = [
                pltpu.async_copy(
                    table_hbm.at[idx_v.at[q0 + q]],
                    buf.at[pl.ds(q * 128, 128)],
                    gsem,
                )
                for q in range(Q)
            ]
            for g in gathers:
                g.wait()

            def sl_body(sl, c2):
                colv = jnp.full((16,), sl, jnp.int32)
                for q in range(Q):
                    row = buf.at[q * 128 + sl]
                    for m in range(D // 16):
                        vec = row[pl.ds(16 * m, 16)] * SCALE
                        plsc.store_scatter(tbuf, [rowvecs[q][m], colv], vec)
                return c2

            lax.fori_loop(0, 128, sl_body, 0)

            stores = []
            for q in range(Q):
                gid = gid0 + q0 + q
                p_lo = gid % 8
                ph_sh = gid // 8
                sh = ph_sh % sh_n
                p = 8 * (ph_sh // sh_n) + p_lo
                cb = p * (D // 8) * sh_n + sh
                for g in range(D // 8):
                    stores.append(
                        pltpu.async_copy(
                            tbuf.at[pl.ds(q * D + 8 * g, 8), pl.ds(0, 128)],
                            out_hbm.at[cb + g * sh_n],
                            ssem,
                        )
                    )
            for st in stores:
                st.wait()
            return carry

        lax.fori_loop(0, n_iters, it_body, 0)

    return emb


def kernel(tokens, table):
    S, P = tokens.shape
    V, D = table.shape
    tok2 = (
        tokens.T.reshape(P // 8, 8, S // 128, 128)
        .transpose(0, 2, 1, 3)
        .reshape((S // 128) * P, 128)
        .astype(jnp.int32)
    )
    a2 = _build(S, P, V, D)(tok2, table)
    out = (
        a2.reshape(P, D // 8, S // 128, 8, 128)
        .transpose(2, 4, 0, 1, 3)
        .reshape(S, P, D)
    )
    return out
